# Initial kernel scaffold; baseline (speedup 1.0000x reference)
import jax
import jax.numpy as jnp
from jax.experimental import pallas as pl


def kernel(x, in_degree, out_degree, atom_w, in_deg_w, out_deg_w, graph_token_w):
    raise NotImplementedError("write your pallas kernel here")


# speedup vs baseline: 6.8403x; 6.8403x over previous
import functools

import jax
import jax.numpy as jnp
from jax import lax
from jax.experimental import pallas as pl
from jax.experimental.pallas import tpu as pltpu
from jax.experimental.pallas import tpu_sc as plsc

_NC, _NS = 2, 16
_NW = _NC * _NS
_B, _N, _F = 256, 128, 9
_FANIN = _F + 2
_H = 128
_CHUNK = 32
_NCHUNK = _N // _CHUNK
_GPW = _B // _NW
_ROWS_OUT = _B * (_N + 1)


def _sc_embed(table, idx, token):
    mesh = plsc.VectorSubcoreMesh(core_axis_name="c", subcore_axis_name="s",
                                  num_cores=_NC, num_subcores=_NS)

    @functools.partial(
        pl.kernel,
        out_type=jax.ShapeDtypeStruct((_ROWS_OUT, _H), jnp.float32),
        mesh=mesh,
        compiler_params=pltpu.CompilerParams(use_tc_tiling_on_sc=False),
        scratch_types=[
            pltpu.VMEM((_NCHUNK, _FANIN, _CHUNK), jnp.int32),
            pltpu.VMEM((_FANIN, _CHUNK, _H), jnp.float32),
            pltpu.VMEM((_CHUNK, _H), jnp.float32),
            pltpu.VMEM((1, _H), jnp.float32),
            pltpu.SemaphoreType.DMA,
        ],
    )
    def k(table_hbm, idx_hbm, token_hbm, out_hbm, idx_v, gath, outc, tok_v, sem):
        wid = lax.axis_index("s") * _NC + lax.axis_index("c")
        pltpu.sync_copy(token_hbm, tok_v)

        @pl.loop(0, _GPW)
        def _graph(gl):
            g = wid * _GPW + gl
            pltpu.sync_copy(idx_hbm.at[g], idx_v)
            pltpu.sync_copy(tok_v, out_hbm.at[pl.ds(g * (_N + 1), 1)])

            @pl.loop(0, _NCHUNK)
            def _chunk(c):
                for j in range(_FANIN):
                    pltpu.async_copy(table_hbm.at[idx_v.at[c, j]], gath.at[j], sem)
                for j in range(_FANIN):
                    pltpu.make_async_copy(
                        table_hbm.at[idx_v.at[c, j]], gath.at[j], sem).wait()

                @pl.loop(0, _CHUNK)
                def _node(i):
                    for col in range(_H // 16):
                        sl = pl.ds(col * 16, 16)
                        acc = gath[0, i, sl]
                        for j in range(1, _FANIN):
                            acc = acc + gath[j, i, sl]
                        outc[i, sl] = acc

                pltpu.sync_copy(
                    outc, out_hbm.at[pl.ds(g * (_N + 1) + 1 + c * _CHUNK, _CHUNK)])

    return k(table, idx, token)


def kernel(x, in_degree, out_degree, atom_w, in_deg_w, out_deg_w, graph_token_w):
    table = jnp.concatenate([atom_w, in_deg_w, out_deg_w], axis=0)
    na = atom_w.shape[0]
    ndi = in_deg_w.shape[0]
    xi = x.astype(jnp.int32)
    ii = (in_degree + na).astype(jnp.int32)
    oi = (out_degree + na + ndi).astype(jnp.int32)
    allidx = jnp.concatenate([xi, ii[..., None], oi[..., None]], axis=-1)
    idx = allidx.reshape(_B, _NCHUNK, _CHUNK, _FANIN).transpose(0, 1, 3, 2)
    out = _sc_embed(table, idx, graph_token_w)
    return out.reshape(_B, _N + 1, _H)
